# baseline (speedup 1.0000x reference)
import functools

import jax
import jax.numpy as jnp
from jax import lax
from jax.experimental import pallas as pl
from jax.experimental.pallas import tpu as pltpu
from jax.experimental.pallas import tpu_sc as plsc

B = 16384
C = 4
E = 64
NC, NS, L = 2, 16, 16
NW = NC * NS
BPW = B // NW
CB = 64
NCHUNK = BPW // CB
MAXG = 128


def _sc_body(t_idx_hbm, c_idx_hbm, t_tab, c_tab, out_hbm,
             t_idx_v, c_idx_v, t_rows, c_rows, out_v, sem0, sem1):
    wid = lax.axis_index("s") * NC + lax.axis_index("c")
    base = wid * BPW

    pltpu.sync_copy(t_idx_hbm.at[pl.ds(base, BPW)], t_idx_v)
    pltpu.sync_copy(c_idx_hbm.at[pl.ds(base * C, BPW * C)], c_idx_v)

    sems = (sem0, sem1)
    lanes = lax.broadcasted_iota(jnp.int32, (L,), 0)

    def issue(g, buf):
        off = g * CB
        cps = [pltpu.async_copy(
            t_tab.at[t_idx_v.at[pl.ds(off, CB)]], t_rows.at[buf], sems[buf])]
        for j in range(CB * C // MAXG):
            cps.append(pltpu.async_copy(
                c_tab.at[c_idx_v.at[pl.ds(off * C + j * MAXG, MAXG)]],
                c_rows.at[buf, pl.ds(j * MAXG, MAXG)], sems[buf]))
        return cps

    def compute(g, buf):
        tr = t_rows.at[buf]
        cr = c_rows.at[buf]
        for grp in range(CB // L):
            brow = grp * L + lanes
            def e_body(e, accs):
                ecol = jnp.full((L,), e, jnp.int32)
                tv = plsc.load_gather(tr, [brow, ecol])
                return tuple(
                    acc + tv * plsc.load_gather(cr, [brow * C + c, ecol])
                    for c, acc in enumerate(accs))
            accs = lax.fori_loop(
                0, E, e_body, tuple(jnp.zeros((L,), jnp.float32)
                                    for _ in range(C)))
            for c in range(C):
                plsc.store_scatter(out_v, [brow * C + c], accs[c])
        pltpu.sync_copy(out_v, out_hbm.at[pl.ds((base + g * CB) * C, CB * C)])

    pend = issue(0, 0)
    for g in range(NCHUNK):
        nxt = issue(g + 1, (g + 1) % 2) if g + 1 < NCHUNK else None
        for cp in pend:
            cp.wait()
        compute(g, g % 2)
        pend = nxt


@jax.jit
def kernel(target, context, target_table, context_table):
    if target.ndim == 2:
        target = jnp.squeeze(target, axis=1)
    mesh = plsc.VectorSubcoreMesh(core_axis_name="c", subcore_axis_name="s")
    run = pl.kernel(
        _sc_body,
        out_type=jax.ShapeDtypeStruct((B * C,), jnp.float32),
        mesh=mesh,
        scratch_types=[
            pltpu.VMEM((BPW,), jnp.int32),
            pltpu.VMEM((BPW * C,), jnp.int32),
            pltpu.VMEM((2, CB, E), jnp.float32),
            pltpu.VMEM((2, CB * C, E), jnp.float32),
            pltpu.VMEM((CB * C,), jnp.float32),
            pltpu.SemaphoreType.DMA,
            pltpu.SemaphoreType.DMA,
        ],
        compiler_params=pltpu.CompilerParams(
            needs_layout_passes=False, use_tc_tiling_on_sc=False),
    )
    out = run(target.astype(jnp.int32), context.astype(jnp.int32).reshape(-1),
              target_table, context_table)
    return out.reshape(B, C)
